# Initial kernel scaffold; baseline (speedup 1.0000x reference)
import functools

import jax
import jax.numpy as jnp
from jax import lax
from jax.experimental import pallas as pl
from jax.experimental.pallas import tpu as pltpu
from jax.experimental.pallas import tpu_sc as plsc

N_NODES = 10000
N_EDGES = 160000
H_DIM = 256
NUM_RELS = 8
NUM_BASES = 4
N_POS = 32768
N_NEG = 32768

NC = 2
NS = 16
L = 16

HALF = N_NODES // NC
ROWS_PER_TILE = 320
ACC_ROWS = NS * ROWS_PER_TILE
TRASH = HALF
CNT_COLS = 16

EDGES_PER_TILE = N_EDGES // NS
CE = 80
N_CHUNKS = EDGES_PER_TILE // CE

_mesh = plsc.VectorSubcoreMesh(
    core_axis_name="c", subcore_axis_name="s", num_cores=NC, num_subcores=NS
)


@functools.partial(
    pl.kernel,
    out_type=jax.ShapeDtypeStruct((NC, ACC_ROWS, CNT_COLS), jnp.float32),
    mesh=_mesh,
    scratch_types=[
        pltpu.VMEM_SHARED((ACC_ROWS, CNT_COLS), jnp.float32),
        pltpu.VMEM((ROWS_PER_TILE, CNT_COLS), jnp.float32),
        pltpu.VMEM((CE,), jnp.int32),
        pltpu.VMEM((CE,), jnp.int32),
        pltpu.VMEM((CE,), jnp.int32),
        pltpu.VMEM((CE, CNT_COLS), jnp.float32),
    ],
)
def _sc_invcnt(dst_hbm, et_hbm, out_hbm, cnt_sh, buf, dstv, etv, ldstv, onehot):
    cid = lax.axis_index("c")
    sid = lax.axis_index("s")
    base = cid * HALF
    zero16 = jnp.zeros((L,), jnp.float32)
    one16 = jnp.ones((L,), jnp.float32)

    def zrow(i, _):
        buf[i, :] = zero16
        return 0

    lax.fori_loop(0, ROWS_PER_TILE, zrow, 0)
    pltpu.sync_copy(buf, cnt_sh.at[pl.ds(sid * ROWS_PER_TILE, ROWS_PER_TILE)])

    def zoh(i, _):
        onehot[i, :] = zero16
        return 0

    lax.fori_loop(0, CE, zoh, 0)
    plsc.subcore_barrier()

    ebase = sid * EDGES_PER_TILE

    def chunk(c, _):
        off = ebase + c * CE
        pltpu.sync_copy(dst_hbm.at[pl.ds(off, CE)], dstv)
        pltpu.sync_copy(et_hbm.at[pl.ds(off, CE)], etv)
        for g in range(CE // L):
            d = dstv[pl.ds(g * L, L)]
            e = etv[pl.ds(g * L, L)]
            ld = d - base
            inr = (d >= base) & (ld < HALF)
            ldstv[pl.ds(g * L, L)] = jnp.where(inr, ld, TRASH)
            rows = lax.iota(jnp.int32, (L,)) + g * L
            plsc.store_scatter(onehot, [rows, e], one16)
        pltpu.sync_copy(onehot, cnt_sh.at[ldstv], add=True)
        for g in range(CE // L):
            e = etv[pl.ds(g * L, L)]
            rows = lax.iota(jnp.int32, (L,)) + g * L
            plsc.store_scatter(onehot, [rows, e], zero16)
        return 0

    lax.fori_loop(0, N_CHUNKS, chunk, 0)
    plsc.subcore_barrier()

    r0 = sid * ROWS_PER_TILE
    pltpu.sync_copy(cnt_sh.at[pl.ds(r0, ROWS_PER_TILE)], buf)

    def inv(i, _):
        v = buf[i, :]
        buf[i, :] = one16 / jnp.maximum(v, one16)
        return 0

    lax.fori_loop(0, ROWS_PER_TILE, inv, 0)
    pltpu.sync_copy(buf, out_hbm.at[cid, pl.ds(r0, ROWS_PER_TILE)])


def _make_sc_layer(rel_stride):
    @functools.partial(
        pl.kernel,
        out_type=jax.ShapeDtypeStruct((NC, ACC_ROWS, H_DIM), jnp.float32),
        mesh=_mesh,
        scratch_types=[
            pltpu.VMEM_SHARED((ACC_ROWS, H_DIM), jnp.float32),
            pltpu.VMEM((ACC_ROWS, CNT_COLS), jnp.float32),
            pltpu.VMEM((CE, H_DIM), jnp.float32),
            pltpu.VMEM((CE,), jnp.int32),
            pltpu.VMEM((CE,), jnp.int32),
            pltpu.VMEM((CE,), jnp.int32),
            pltpu.VMEM((CE,), jnp.int32),
            pltpu.VMEM((CE,), jnp.int32),
            pltpu.VMEM((CE,), jnp.float32),
            pltpu.SemaphoreType.DMA,
        ],
    )
    def _sc_layer(
        table_hbm, src_hbm, dst_hbm, et_hbm, invcnt_hbm, out_hbm,
        acc_sh, invcnt_v, rows_v, srcv, dstv, etv, gidxv, ldstv, wv, sem,
    ):
        cid = lax.axis_index("c")
        sid = lax.axis_index("s")
        base = cid * HALF
        zero16 = jnp.zeros((L,), jnp.float32)

        def zrow(i, _):
            for k in range(H_DIM // L):
                rows_v[i, pl.ds(k * L, L)] = zero16
            return 0

        lax.fori_loop(0, CE, zrow, 0)
        for q in range(ROWS_PER_TILE // CE):
            pltpu.sync_copy(
                rows_v, acc_sh.at[pl.ds(sid * ROWS_PER_TILE + q * CE, CE)]
            )
        pltpu.sync_copy(invcnt_hbm.at[cid], invcnt_v)
        plsc.subcore_barrier()

        ebase = sid * EDGES_PER_TILE

        def chunk(c, _):
            off = ebase + c * CE
            pltpu.sync_copy(src_hbm.at[pl.ds(off, CE)], srcv)
            pltpu.sync_copy(dst_hbm.at[pl.ds(off, CE)], dstv)
            pltpu.sync_copy(et_hbm.at[pl.ds(off, CE)], etv)
            for g in range(CE // L):
                sl = pl.ds(g * L, L)
                s_ = srcv[sl]
                d = dstv[sl]
                e = etv[sl]
                gidxv[sl] = s_ + e * rel_stride
                ld = d - base
                inr = (d >= base) & (ld < HALF)
                ldstv[sl] = jnp.where(inr, ld, TRASH)
                lds = jnp.where(inr, ld, 0)
                wv[sl] = plsc.load_gather(invcnt_v, [lds, e])
            pltpu.async_copy(table_hbm.at[gidxv], rows_v, sem).wait()

            def scale(i, _):
                wi = wv[i]
                for k in range(H_DIM // L):
                    sl2 = pl.ds(k * L, L)
                    rows_v[i, sl2] = rows_v[i, sl2] * wi
                return 0

            lax.fori_loop(0, CE, scale, 0)
            pltpu.sync_copy(rows_v, acc_sh.at[ldstv], add=True)
            return 0

        lax.fori_loop(0, N_CHUNKS, chunk, 0)
        plsc.subcore_barrier()

        for q in range(ROWS_PER_TILE // CE):
            r0 = sid * ROWS_PER_TILE + q * CE
            pltpu.sync_copy(acc_sh.at[pl.ds(r0, CE)], rows_v)

            def rl(i, _):
                for k in range(H_DIM // L):
                    sl2 = pl.ds(k * L, L)
                    rows_v[i, sl2] = jnp.maximum(rows_v[i, sl2], 0.0)
                return 0

            lax.fori_loop(0, CE, rl, 0)
            pltpu.sync_copy(rows_v, out_hbm.at[cid, pl.ds(r0, CE)])

    return _sc_layer


_sc_layer0 = _make_sc_layer(0)
_sc_layer1 = _make_sc_layer(N_NODES)

G_TOT = 2 * N_POS + 2 * N_NEG
GB = 128
ROWS_PER_W = G_TOT // (NC * NS)


@functools.partial(
    pl.kernel,
    out_type=jax.ShapeDtypeStruct((G_TOT, H_DIM), jnp.float32),
    mesh=_mesh,
    scratch_types=[
        pltpu.VMEM((GB,), jnp.int32),
        pltpu.VMEM((GB, H_DIM), jnp.float32),
        pltpu.SemaphoreType.DMA,
    ],
)
def _sc_gather(table_hbm, idx_hbm, out_hbm, idxv, rows, sem):
    cid = lax.axis_index("c")
    sid = lax.axis_index("s")
    wid = cid * NS + sid

    def chunk(c, _):
        off = wid * ROWS_PER_W + c * GB
        pltpu.sync_copy(idx_hbm.at[pl.ds(off, GB)], idxv)
        pltpu.async_copy(table_hbm.at[idxv], rows, sem).wait()
        pltpu.sync_copy(rows, out_hbm.at[pl.ds(off, GB)])
        return 0

    lax.fori_loop(0, ROWS_PER_W // GB, chunk, 0)


def _tc_transform(h, basis, coeff):
    BN = 400

    def body(coeff_ref, h_ref, basis_ref, out_ref):
        r = pl.program_id(0)
        w = coeff_ref[r, 0] * basis_ref[0]
        for b in range(1, NUM_BASES):
            w = w + coeff_ref[r, b] * basis_ref[b]
        out_ref[0] = jnp.dot(h_ref[...], w, preferred_element_type=jnp.float32)

    return pl.pallas_call(
        body,
        grid=(NUM_RELS, N_NODES // BN),
        in_specs=[
            pl.BlockSpec(memory_space=pltpu.SMEM),
            pl.BlockSpec((BN, H_DIM), lambda r, j: (j, 0)),
            pl.BlockSpec((NUM_BASES, H_DIM, H_DIM), lambda r, j: (0, 0, 0)),
        ],
        out_specs=pl.BlockSpec((1, BN, H_DIM), lambda r, j: (r, j, 0)),
        out_shape=jax.ShapeDtypeStruct((NUM_RELS, N_NODES, H_DIM), jnp.float32),
    )(coeff, h, basis)


def kernel(embed, basis, coeff, w_relation, edge_index, edge_type,
           p_edge_index, p_etype, n_edge_index):
    src = edge_index[0].astype(jnp.int32)
    dst = edge_index[1].astype(jnp.int32)
    et = edge_type.astype(jnp.int32)

    invcnt = _sc_invcnt(dst, et)

    out0 = _sc_layer0(embed, src, dst, et, invcnt)
    h1 = out0[:, :HALF, :].reshape(N_NODES, H_DIM)

    hh = _tc_transform(h1, basis, coeff)
    table1 = hh.reshape(NUM_RELS * N_NODES, H_DIM)

    out1 = _sc_layer1(table1, src, dst, et, invcnt)
    h2 = out1[:, :HALF, :].reshape(N_NODES, H_DIM)

    gidx = jnp.concatenate(
        [p_edge_index[0], p_edge_index[1], n_edge_index[0], n_edge_index[1]]
    ).astype(jnp.int32)
    rows = _sc_gather(h2, gidx)

    p_head = rows[:N_POS]
    p_tail = rows[N_POS : 2 * N_POS]
    n_head = rows[2 * N_POS : 3 * N_POS]
    n_tail = rows[3 * N_POS :]
    return (p_head, p_tail, p_etype, n_head, n_tail)


# speedup vs baseline: 3.4063x; 3.4063x over previous
import functools

import jax
import jax.numpy as jnp
from jax import lax
from jax.experimental import pallas as pl
from jax.experimental.pallas import tpu as pltpu
from jax.experimental.pallas import tpu_sc as plsc

N_NODES = 10000
N_EDGES = 160000
H_DIM = 256
NUM_RELS = 8
NUM_BASES = 4
N_POS = 32768
N_NEG = 32768

NC = 2
NS = 16
L = 16

HALF = N_NODES // NC
NQ = 4
QUARTER = N_NODES // NQ
QROWS_PER_TILE = 160
ACC_ROWS = NS * QROWS_PER_TILE
TRASH = QUARTER
CNT_COLS = 16
CNT_W = 128
CNT_ROWS_PER_TILE = 42
CNT_ROWS = NS * CNT_ROWS_PER_TILE
CNT_TRASH_ROW = 640

EDGES_PER_TILE = N_EDGES // NS
CE = 80
N_CHUNKS = EDGES_PER_TILE // CE

_mesh = plsc.VectorSubcoreMesh(
    core_axis_name="c", subcore_axis_name="s", num_cores=NC, num_subcores=NS
)


@functools.partial(
    pl.kernel,
    out_type=jax.ShapeDtypeStruct((NC, CNT_ROWS * CNT_W), jnp.float32),
    mesh=_mesh,
    compiler_params=pltpu.CompilerParams(needs_layout_passes=False),
    scratch_types=[
        pltpu.VMEM_SHARED((CNT_ROWS, CNT_W), jnp.float32),
        pltpu.VMEM((CNT_ROWS_PER_TILE, CNT_W), jnp.float32),
        pltpu.VMEM((CNT_ROWS_PER_TILE * CNT_W,), jnp.float32),
        pltpu.VMEM((CE,), jnp.int32),
        pltpu.VMEM((CE,), jnp.int32),
        pltpu.VMEM((CE,), jnp.int32),
        pltpu.VMEM((CE, CNT_W), jnp.float32),
    ],
)
def _sc_invcnt(dst_hbm, et_hbm, out_hbm, cnt_sh, buf, buff, dstv, etv, ldstv, onehot):
    cid = lax.axis_index("c")
    sid = lax.axis_index("s")
    base = cid * HALF
    zero16 = jnp.zeros((L,), jnp.float32)
    one16 = jnp.ones((L,), jnp.float32)

    def zrow(i, _):
        for s in range(CNT_W // L):
            buf[i, pl.ds(s * L, L)] = zero16
        return 0

    lax.fori_loop(0, CNT_ROWS_PER_TILE, zrow, 0)
    pltpu.sync_copy(
        buf, cnt_sh.at[pl.ds(sid * CNT_ROWS_PER_TILE, CNT_ROWS_PER_TILE)]
    )
    plsc.subcore_barrier()

    ebase = sid * EDGES_PER_TILE
    lane_consts = [lax.iota(jnp.int32, L) + s * L for s in range(CNT_W // L)]

    def chunk(c, _):
        off = ebase + c * CE
        pltpu.sync_copy(dst_hbm.at[pl.ds(off, CE)], dstv)
        pltpu.sync_copy(et_hbm.at[pl.ds(off, CE)], etv)
        for g in range(CE // L):
            d = dstv[pl.ds(g * L, L)]
            e = etv[pl.ds(g * L, L)]
            ld = d - base
            inr = (d >= base) & (ld < HALF)
            f = jnp.where(inr, ld * CNT_COLS + e, CNT_TRASH_ROW * CNT_W)
            ldstv[pl.ds(g * L, L)] = lax.shift_right_logical(f, 7)
            col = lax.bitwise_and(f, 127)
            cs = [col[j] for j in range(L)]
            for j in range(L):
                for s in range(CNT_W // L):
                    onehot[g * L + j, pl.ds(s * L, L)] = jnp.where(
                        lane_consts[s] == cs[j], jnp.float32(1.0), jnp.float32(0.0)
                    )
        pltpu.sync_copy(onehot, cnt_sh.at[ldstv], add=True)
        return 0

    lax.fori_loop(0, N_CHUNKS, chunk, 0)
    plsc.subcore_barrier()

    r0 = sid * CNT_ROWS_PER_TILE
    pltpu.sync_copy(cnt_sh.at[pl.ds(r0, CNT_ROWS_PER_TILE)], buf)

    def inv(i, _):
        for s in range(CNT_W // L):
            v = buf[i, pl.ds(s * L, L)]
            buff[pl.ds(i * CNT_W + s * L, L)] = one16 / jnp.maximum(v, one16)
        return 0

    lax.fori_loop(0, CNT_ROWS_PER_TILE, inv, 0)
    pltpu.sync_copy(
        buff,
        out_hbm.at[cid, pl.ds(r0 * CNT_W, CNT_ROWS_PER_TILE * CNT_W)],
    )


def _make_sc_layer(rel_stride):
    HW = H_DIM // 2

    @functools.partial(
        pl.kernel,
        out_type=jax.ShapeDtypeStruct((NC, 2, 2, ACC_ROWS, HW), jnp.float32),
        mesh=_mesh,
        compiler_params=pltpu.CompilerParams(needs_layout_passes=False),
        scratch_types=[
            pltpu.VMEM_SHARED((ACC_ROWS, HW), jnp.float32),
            pltpu.VMEM((CNT_ROWS * CNT_W,), jnp.float32),
            pltpu.VMEM((CE, HW), jnp.float32),
            pltpu.VMEM((CE,), jnp.int32),
            pltpu.VMEM((CE,), jnp.int32),
            pltpu.VMEM((CE,), jnp.int32),
            pltpu.VMEM((CE,), jnp.int32),
            pltpu.VMEM((CE,), jnp.int32),
            pltpu.VMEM((CE,), jnp.float32),
            pltpu.SemaphoreType.DMA,
        ],
    )
    def _sc_layer(
        table_hbm, src_hbm, dst_hbm, et_hbm, invcnt_hbm, out_hbm,
        acc_sh, invcnt_v, rows_v, srcv, dstv, etv, gidxv, ldstv, wv, sem,
    ):
        cid = lax.axis_index("c")
        sid = lax.axis_index("s")
        hbase = cid * HALF
        zero16 = jnp.zeros((L,), jnp.float32)

        pltpu.sync_copy(invcnt_hbm.at[cid], invcnt_v)
        ebase = sid * EDGES_PER_TILE

        for p in range(2):
            qbase = hbase + p * QUARTER
            for h in range(2):

                def zrow(i, _):
                    for k in range(HW // L):
                        rows_v[i, pl.ds(k * L, L)] = zero16
                    return 0

                lax.fori_loop(0, CE, zrow, 0)
                for q in range(QROWS_PER_TILE // CE):
                    pltpu.sync_copy(
                        rows_v,
                        acc_sh.at[pl.ds(sid * QROWS_PER_TILE + q * CE, CE)],
                    )
                plsc.subcore_barrier()

                def chunk(c, _):
                    off = ebase + c * CE
                    pltpu.sync_copy(src_hbm.at[pl.ds(off, CE)], srcv)
                    pltpu.sync_copy(dst_hbm.at[pl.ds(off, CE)], dstv)
                    pltpu.sync_copy(et_hbm.at[pl.ds(off, CE)], etv)
                    for g in range(CE // L):
                        sl = pl.ds(g * L, L)
                        s_ = srcv[sl]
                        d = dstv[sl]
                        e = etv[sl]
                        gidxv[sl] = s_ * 2 + e * (2 * rel_stride) + h
                        ld = d - qbase
                        inr = (d >= qbase) & (ld < QUARTER)
                        ldstv[sl] = jnp.where(inr, ld, TRASH)
                        ldh = d - hbase
                        widx = jnp.where(inr, ldh * CNT_COLS + e, 0)
                        wv[sl] = plsc.load_gather(invcnt_v, [widx])
                    pltpu.async_copy(table_hbm.at[gidxv], rows_v, sem).wait()

                    for g in range(CE // L):
                        w16 = wv[pl.ds(g * L, L)]
                        ws = [w16[j] for j in range(L)]

                        def scale(k, _):
                            sl2 = pl.ds(k * L, L)
                            for j in range(L):
                                rows_v[g * L + j, sl2] = (
                                    rows_v[g * L + j, sl2] * ws[j]
                                )
                            return 0

                        lax.fori_loop(0, HW // L, scale, 0)
                    pltpu.sync_copy(rows_v, acc_sh.at[ldstv], add=True)
                    return 0

                lax.fori_loop(0, N_CHUNKS, chunk, 0)
                plsc.subcore_barrier()

                for q in range(QROWS_PER_TILE // CE):
                    r0 = sid * QROWS_PER_TILE + q * CE
                    pltpu.sync_copy(acc_sh.at[pl.ds(r0, CE)], rows_v)

                    def rl(i, _):
                        for k in range(HW // L):
                            sl2 = pl.ds(k * L, L)
                            rows_v[i, sl2] = jnp.maximum(rows_v[i, sl2], 0.0)
                        return 0

                    lax.fori_loop(0, CE, rl, 0)
                    pltpu.sync_copy(rows_v, out_hbm.at[cid, p, h, pl.ds(r0, CE)])
                plsc.subcore_barrier()

    return _sc_layer


_sc_layer0 = _make_sc_layer(0)
_sc_layer1 = _make_sc_layer(N_NODES)

G_TOT = 2 * N_POS + 2 * N_NEG
GB = 128
ROWS_PER_W = G_TOT // (NC * NS)


@functools.partial(
    pl.kernel,
    out_type=jax.ShapeDtypeStruct((G_TOT, H_DIM), jnp.float32),
    mesh=_mesh,
    compiler_params=pltpu.CompilerParams(needs_layout_passes=False),
    scratch_types=[
        pltpu.VMEM((GB,), jnp.int32),
        pltpu.VMEM((GB, H_DIM), jnp.float32),
        pltpu.SemaphoreType.DMA,
    ],
)
def _sc_gather(table_hbm, idx_hbm, out_hbm, idxv, rows, sem):
    cid = lax.axis_index("c")
    sid = lax.axis_index("s")
    wid = cid * NS + sid

    def chunk(c, _):
        off = wid * ROWS_PER_W + c * GB
        pltpu.sync_copy(idx_hbm.at[pl.ds(off, GB)], idxv)
        pltpu.async_copy(table_hbm.at[idxv], rows, sem).wait()
        pltpu.sync_copy(rows, out_hbm.at[pl.ds(off, GB)])
        return 0

    lax.fori_loop(0, ROWS_PER_W // GB, chunk, 0)


def _tc_transform(h, basis, coeff):
    BN = 400

    def body(coeff_ref, h_ref, basis_ref, out_ref):
        r = pl.program_id(0)
        w = coeff_ref[r, 0] * basis_ref[0]
        for b in range(1, NUM_BASES):
            w = w + coeff_ref[r, b] * basis_ref[b]
        out_ref[0] = jnp.dot(h_ref[...], w, preferred_element_type=jnp.float32)

    return pl.pallas_call(
        body,
        grid=(NUM_RELS, N_NODES // BN),
        in_specs=[
            pl.BlockSpec(memory_space=pltpu.SMEM),
            pl.BlockSpec((BN, H_DIM), lambda r, j: (j, 0)),
            pl.BlockSpec((NUM_BASES, H_DIM, H_DIM), lambda r, j: (0, 0, 0)),
        ],
        out_specs=pl.BlockSpec((1, BN, H_DIM), lambda r, j: (r, j, 0)),
        out_shape=jax.ShapeDtypeStruct((NUM_RELS, N_NODES, H_DIM), jnp.float32),
    )(coeff, h, basis)


def kernel(embed, basis, coeff, w_relation, edge_index, edge_type,
           p_edge_index, p_etype, n_edge_index):
    src = edge_index[0].astype(jnp.int32)
    dst = edge_index[1].astype(jnp.int32)
    et = edge_type.astype(jnp.int32)

    invcnt = _sc_invcnt(dst, et)

    emb_h = embed.reshape(2 * N_NODES, H_DIM // 2)
    out0 = _sc_layer0(emb_h, src, dst, et, invcnt)
    h1 = (
        out0[:, :, :, :QUARTER, :]
        .transpose(0, 1, 3, 2, 4)
        .reshape(N_NODES, H_DIM)
    )

    hh = _tc_transform(h1, basis, coeff)
    table1 = hh.reshape(2 * NUM_RELS * N_NODES, H_DIM // 2)

    out1 = _sc_layer1(table1, src, dst, et, invcnt)
    h2 = (
        out1[:, :, :, :QUARTER, :]
        .transpose(0, 1, 3, 2, 4)
        .reshape(N_NODES, H_DIM)
    )

    gidx = jnp.concatenate(
        [p_edge_index[0], p_edge_index[1], n_edge_index[0], n_edge_index[1]]
    ).astype(jnp.int32)
    rows = _sc_gather(h2, gidx)

    p_head = rows[:N_POS]
    p_tail = rows[N_POS : 2 * N_POS]
    n_head = rows[2 * N_POS : 3 * N_POS]
    n_tail = rows[3 * N_POS :]
    return (p_head, p_tail, p_etype, n_head, n_tail)
